# baseline (speedup 1.0000x reference)
import functools

import jax
import jax.numpy as jnp
from jax.experimental import pallas as pl
from jax.experimental.pallas import tpu as pltpu


def _round_up(x: int, m: int) -> int:
    return (x + m - 1) // m * m


def _deferred_epilogue_block(x_ref, w_ref, b_ref, o_ref, acc_ref, *, inv_s, bt, tpc):
    s = pl.program_id(1)
    mean_tok = jnp.sum(x_ref[...], axis=1, dtype=jnp.float32) * inv_s
    acc_ref[pl.ds(s * bt, bt), :] = mean_tok

    @pl.when(s == tpc - 1)
    def _epilogue():
        y = jax.lax.dot_general(
            acc_ref[...].astype(w_ref.dtype), w_ref[...],
            dimension_numbers=(((1,), (1,)), ((), ())),
            preferred_element_type=jnp.float32)
        o_ref[...] = jnp.tanh(y + b_ref[...].astype(jnp.float32)).astype(o_ref.dtype)


def _fused_epilogue_block(x_ref, w_ref, b_ref, o_ref, *, inv_s):
    mean_tok = jnp.sum(x_ref[...], axis=1, dtype=jnp.float32) * inv_s
    y = jax.lax.dot_general(
        mean_tok.astype(w_ref.dtype), w_ref[...],
        dimension_numbers=(((1,), (1,)), ((), ())),
        preferred_element_type=jnp.float32)
    o_ref[...] = jnp.tanh(y + b_ref[...].astype(jnp.float32)).astype(o_ref.dtype)


def kernel(hidden_states, weight, bias):
    B, S, H = hidden_states.shape
    out_dtype = hidden_states.dtype
    x_isz = hidden_states.dtype.itemsize

    row_bytes = S * H * x_isz
    budget = 7 << 20
    Bt = max(8, min(128, (budget // max(1, row_bytes)) // 8 * 8))
    if B <= 8:
        Bt = B
    else:
        Bt = min(Bt, max(8, _round_up(pl.cdiv(B, 4), 8)))
    nb = pl.cdiv(B, Bt)

    bias2d = bias.reshape(1, H)
    cost = pl.CostEstimate(
        flops=int(B * S * H + 2 * B * H * H + B * H),
        transcendentals=int(B * H),
        bytes_accessed=int(hidden_states.size * x_isz + weight.size * 4
                           + bias.size * 4 + B * H * out_dtype.itemsize))

    if nb % 2 == 0 and nb >= 4:
        tpc = nb // 2
        body = functools.partial(_deferred_epilogue_block,
                                 inv_s=1.0 / S, bt=Bt, tpc=tpc)
        return pl.pallas_call(
            body,
            out_shape=jax.ShapeDtypeStruct((B, H), out_dtype),
            grid=(2, tpc),
            in_specs=[
                pl.BlockSpec((Bt, S, H), lambda c, s: (c * tpc + s, 0, 0)),
                pl.BlockSpec((H, H), lambda c, s: (0, 0)),
                pl.BlockSpec((1, H), lambda c, s: (0, 0)),
            ],
            out_specs=pl.BlockSpec((Bt * tpc, H), lambda c, s: (c, 0)),
            scratch_shapes=[pltpu.VMEM((Bt * tpc, H), jnp.float32)],
            compiler_params=pltpu.CompilerParams(
                dimension_semantics=("parallel", "arbitrary")),
            cost_estimate=cost,
        )(hidden_states, weight, bias2d)

    body = functools.partial(_fused_epilogue_block, inv_s=1.0 / S)
    return pl.pallas_call(
        body,
        out_shape=jax.ShapeDtypeStruct((B, H), out_dtype),
        grid=(nb,),
        in_specs=[
            pl.BlockSpec((Bt, S, H), lambda b: (b, 0, 0)),
            pl.BlockSpec((H, H), lambda b: (0, 0)),
            pl.BlockSpec((1, H), lambda b: (0, 0)),
        ],
        out_specs=pl.BlockSpec((Bt, H), lambda b: (b, 0)),
        compiler_params=pltpu.CompilerParams(
            dimension_semantics=("parallel",)),
        cost_estimate=cost,
    )(hidden_states, weight, bias2d)


# speedup vs baseline: 1.0114x; 1.0114x over previous
import functools

import jax
import jax.numpy as jnp
from jax.experimental import pallas as pl
from jax.experimental.pallas import tpu as pltpu


def _round_up(x: int, m: int) -> int:
    return (x + m - 1) // m * m


def _deferred_epilogue_block(x_ref, w_ref, b_ref, o_ref, acc_ref, *, inv_s, bt, tpc):
    s = pl.program_id(1)
    mean_tok = jnp.sum(x_ref[...], axis=1, dtype=jnp.float32) * inv_s
    acc_ref[pl.ds(s * bt, bt), :] = mean_tok
    n_head = (tpc - 1) * bt

    @pl.when(s == tpc - 2)
    def _head_epilogue():
        y = jax.lax.dot_general(
            acc_ref[pl.ds(0, n_head), :].astype(w_ref.dtype), w_ref[...],
            dimension_numbers=(((1,), (1,)), ((), ())),
            preferred_element_type=jnp.float32)
        o_ref[pl.ds(0, n_head), :] = jnp.tanh(
            y + b_ref[...].astype(jnp.float32)).astype(o_ref.dtype)

    @pl.when(s == tpc - 1)
    def _tail_epilogue():
        y = jax.lax.dot_general(
            mean_tok.astype(w_ref.dtype), w_ref[...],
            dimension_numbers=(((1,), (1,)), ((), ())),
            preferred_element_type=jnp.float32)
        o_ref[pl.ds(n_head, bt), :] = jnp.tanh(
            y + b_ref[...].astype(jnp.float32)).astype(o_ref.dtype)


def _fused_epilogue_block(x_ref, w_ref, b_ref, o_ref, *, inv_s):
    mean_tok = jnp.sum(x_ref[...], axis=1, dtype=jnp.float32) * inv_s
    y = jax.lax.dot_general(
        mean_tok.astype(w_ref.dtype), w_ref[...],
        dimension_numbers=(((1,), (1,)), ((), ())),
        preferred_element_type=jnp.float32)
    o_ref[...] = jnp.tanh(y + b_ref[...].astype(jnp.float32)).astype(o_ref.dtype)


def kernel(hidden_states, weight, bias):
    B, S, H = hidden_states.shape
    out_dtype = hidden_states.dtype
    x_isz = hidden_states.dtype.itemsize

    row_bytes = S * H * x_isz
    budget = 7 << 20
    Bt = max(8, min(128, (budget // max(1, row_bytes)) // 8 * 8))
    if B <= 8:
        Bt = B
    else:
        Bt = min(Bt, max(8, _round_up(pl.cdiv(B, 4), 8)))
    nb = pl.cdiv(B, Bt)

    bias2d = bias.reshape(1, H)
    cost = pl.CostEstimate(
        flops=int(B * S * H + 2 * B * H * H + B * H),
        transcendentals=int(B * H),
        bytes_accessed=int(hidden_states.size * x_isz + weight.size * 4
                           + bias.size * 4 + B * H * out_dtype.itemsize))

    if nb % 2 == 0 and nb >= 4:
        tpc = nb // 2
        body = functools.partial(_deferred_epilogue_block,
                                 inv_s=1.0 / S, bt=Bt, tpc=tpc)
        return pl.pallas_call(
            body,
            out_shape=jax.ShapeDtypeStruct((B, H), out_dtype),
            grid=(2, tpc),
            in_specs=[
                pl.BlockSpec((Bt, S, H), lambda c, s: (c * tpc + s, 0, 0)),
                pl.BlockSpec((H, H), lambda c, s: (0, 0)),
                pl.BlockSpec((1, H), lambda c, s: (0, 0)),
            ],
            out_specs=pl.BlockSpec((Bt * tpc, H), lambda c, s: (c, 0)),
            scratch_shapes=[pltpu.VMEM((Bt * tpc, H), jnp.float32)],
            compiler_params=pltpu.CompilerParams(
                dimension_semantics=("parallel", "arbitrary")),
            cost_estimate=cost,
        )(hidden_states, weight, bias2d)

    body = functools.partial(_fused_epilogue_block, inv_s=1.0 / S)
    return pl.pallas_call(
        body,
        out_shape=jax.ShapeDtypeStruct((B, H), out_dtype),
        grid=(nb,),
        in_specs=[
            pl.BlockSpec((Bt, S, H), lambda b: (b, 0, 0)),
            pl.BlockSpec((H, H), lambda b: (0, 0)),
            pl.BlockSpec((1, H), lambda b: (0, 0)),
        ],
        out_specs=pl.BlockSpec((Bt, H), lambda b: (b, 0)),
        compiler_params=pltpu.CompilerParams(
            dimension_semantics=("parallel",)),
        cost_estimate=cost,
    )(hidden_states, weight, bias2d)


# speedup vs baseline: 1.0122x; 1.0008x over previous
import functools

import jax
import jax.numpy as jnp
from jax.experimental import pallas as pl
from jax.experimental.pallas import tpu as pltpu


def _round_up(x: int, m: int) -> int:
    return (x + m - 1) // m * m


def _deferred_epilogue_block(x_ref, w_ref, b_ref, o_ref, acc_ref, w16_ref,
                             *, inv_s, bt, tpc):
    s = pl.program_id(1)
    mean_tok = jnp.sum(x_ref[...], axis=1, dtype=jnp.float32) * inv_s
    acc_ref[pl.ds(s * bt, bt), :] = mean_tok
    n_head = (tpc - 1) * bt

    @pl.when(s == 0)
    def _prep_w16():
        w16_ref[...] = w_ref[...].astype(jnp.bfloat16)

    @pl.when(s == tpc - 2)
    def _head_epilogue():
        y = jax.lax.dot_general(
            acc_ref[pl.ds(0, n_head), :].astype(w_ref.dtype), w_ref[...],
            dimension_numbers=(((1,), (1,)), ((), ())),
            preferred_element_type=jnp.float32)
        o_ref[pl.ds(0, n_head), :] = jnp.tanh(
            y + b_ref[...].astype(jnp.float32)).astype(o_ref.dtype)

    @pl.when(s == tpc - 1)
    def _tail_epilogue():
        y = jax.lax.dot_general(
            mean_tok.astype(jnp.bfloat16), w16_ref[...],
            dimension_numbers=(((1,), (1,)), ((), ())),
            preferred_element_type=jnp.float32)
        o_ref[pl.ds(n_head, bt), :] = jnp.tanh(
            y + b_ref[...].astype(jnp.float32)).astype(o_ref.dtype)


def _fused_epilogue_block(x_ref, w_ref, b_ref, o_ref, *, inv_s):
    mean_tok = jnp.sum(x_ref[...], axis=1, dtype=jnp.float32) * inv_s
    y = jax.lax.dot_general(
        mean_tok.astype(w_ref.dtype), w_ref[...],
        dimension_numbers=(((1,), (1,)), ((), ())),
        preferred_element_type=jnp.float32)
    o_ref[...] = jnp.tanh(y + b_ref[...].astype(jnp.float32)).astype(o_ref.dtype)


def kernel(hidden_states, weight, bias):
    B, S, H = hidden_states.shape
    out_dtype = hidden_states.dtype
    x_isz = hidden_states.dtype.itemsize

    row_bytes = S * H * x_isz
    budget = 7 << 20
    Bt = max(8, min(128, (budget // max(1, row_bytes)) // 8 * 8))
    if B <= 8:
        Bt = B
    else:
        Bt = min(Bt, max(8, _round_up(pl.cdiv(B, 4), 8)))
    nb = pl.cdiv(B, Bt)

    bias2d = bias.reshape(1, H)
    cost = pl.CostEstimate(
        flops=int(B * S * H + 2 * B * H * H + B * H),
        transcendentals=int(B * H),
        bytes_accessed=int(hidden_states.size * x_isz + weight.size * 4
                           + bias.size * 4 + B * H * out_dtype.itemsize))

    if nb % 2 == 0 and nb >= 4:
        tpc = nb // 2
        body = functools.partial(_deferred_epilogue_block,
                                 inv_s=1.0 / S, bt=Bt, tpc=tpc)
        return pl.pallas_call(
            body,
            out_shape=jax.ShapeDtypeStruct((B, H), out_dtype),
            grid=(2, tpc),
            in_specs=[
                pl.BlockSpec((Bt, S, H), lambda c, s: (c * tpc + s, 0, 0)),
                pl.BlockSpec((H, H), lambda c, s: (0, 0)),
                pl.BlockSpec((1, H), lambda c, s: (0, 0)),
            ],
            out_specs=pl.BlockSpec((Bt * tpc, H), lambda c, s: (c, 0)),
            scratch_shapes=[pltpu.VMEM((Bt * tpc, H), jnp.float32),
                            pltpu.VMEM((H, H), jnp.bfloat16)],
            compiler_params=pltpu.CompilerParams(
                dimension_semantics=("parallel", "arbitrary")),
            cost_estimate=cost,
        )(hidden_states, weight, bias2d)

    body = functools.partial(_fused_epilogue_block, inv_s=1.0 / S)
    return pl.pallas_call(
        body,
        out_shape=jax.ShapeDtypeStruct((B, H), out_dtype),
        grid=(nb,),
        in_specs=[
            pl.BlockSpec((Bt, S, H), lambda b: (b, 0, 0)),
            pl.BlockSpec((H, H), lambda b: (0, 0)),
            pl.BlockSpec((1, H), lambda b: (0, 0)),
        ],
        out_specs=pl.BlockSpec((Bt, H), lambda b: (b, 0)),
        compiler_params=pltpu.CompilerParams(
            dimension_semantics=("parallel",)),
        cost_estimate=cost,
    )(hidden_states, weight, bias2d)
